# baseline (speedup 1.0000x reference)
import functools
import math

import jax
import jax.numpy as jnp
from jax import lax
from jax.experimental import pallas as pl
from jax.experimental.pallas import tpu as pltpu
from jax.experimental.pallas import tpu_sc as plsc


TM = 512
KB = 512
GATE_TB = 512


def _gate_experts(xf, Wg, bg):
    gate_logits = xf @ Wg.T + bg
    _, top_idx = lax.top_k(gate_logits, 2)
    return jnp.max(top_idx, axis=-1).astype(jnp.int32)


def _sc_gather_rows(table, idx):
    v, d = table.shape
    b = idx.shape[0]
    info = plsc.get_sparse_core_info()
    nw = info.num_cores * info.num_subcores
    assert b % (8 * nw) == 0 and d % info.num_lanes == 0
    b_per_w = b // nw
    ch = 32
    assert b_per_w % ch == 0
    mesh = plsc.VectorSubcoreMesh(core_axis_name="c", subcore_axis_name="s")

    @functools.partial(
        pl.kernel,
        mesh=mesh,
        out_type=jax.ShapeDtypeStruct((b, d), table.dtype),
        scratch_types=[
            pltpu.VMEM((ch,), jnp.int32),
            pltpu.VMEM((ch, d), table.dtype),
            pltpu.SemaphoreType.DMA,
        ],
    )
    def gather(table_hbm, idx_hbm, out_hbm, idx_v, rows_v, sem):
        wid = lax.axis_index("s") * info.num_cores + lax.axis_index("c")
        base = wid * b_per_w
        for c in range(b_per_w // ch):
            off = base + c * ch
            pltpu.sync_copy(idx_hbm.at[pl.ds(off, ch)], idx_v)
            pltpu.async_copy(table_hbm.at[idx_v], rows_v, sem).wait()
            pltpu.sync_copy(rows_v, out_hbm.at[pl.ds(off, ch)])

    return gather(table, idx)


def _ffn_body(meta_ref, xs_ref, w1_ref, w2_ref, b1_ref, g_ref, lb_ref,
              b2_ref, wg_ref, bg_ref, rs_ref, ys_ref, h_ref, xb_ref,
              st_ref, *, k_steps, t_max, dff):
    t = pl.program_id(0)
    s = pl.program_id(1)
    active = t < meta_ref[t_max]

    @pl.when(active & (s == 0))
    def _cast_x():
        xb_ref[...] = xs_ref[...].astype(jnp.bfloat16)

    @pl.when(active & (s < k_steps))
    def _phase1():
        w1 = w1_ref[0].astype(jnp.bfloat16)
        h = lax.dot_general(xb_ref[...], w1, (((1,), (1,)), ((), ())),
                            preferred_element_type=jnp.float32)
        h = h + b1_ref[0, 0]
        h_ref[s] = h.astype(jnp.bfloat16)
        rsum = jnp.sum(h, axis=-1, keepdims=True)
        rsq = jnp.sum(h * h, axis=-1, keepdims=True)

        @pl.when(s == 0)
        def _():
            st_ref[:, 0:1] = rsum
            st_ref[:, 1:2] = rsq

        @pl.when(s > 0)
        def _():
            st_ref[:, 0:1] = st_ref[:, 0:1] + rsum
            st_ref[:, 1:2] = st_ref[:, 1:2] + rsq

    @pl.when(active & (s >= k_steps))
    def _phase2():
        k = s - k_steps
        mu = st_ref[:, 0:1] / dff
        var = st_ref[:, 1:2] / dff - mu * mu
        hn = (h_ref[k].astype(jnp.float32) - mu) / jnp.sqrt(var + 1e-5)
        hn = hn * g_ref[0, 0] + lb_ref[0, 0]
        c0 = math.sqrt(2.0 / math.pi)
        gh = 0.5 * hn * (1.0 + jnp.tanh(c0 * (hn + 0.044715 * hn * hn * hn)))
        w2 = w2_ref[0].astype(jnp.bfloat16)
        part = lax.dot_general(gh.astype(jnp.bfloat16), w2,
                               (((1,), (1,)), ((), ())),
                               preferred_element_type=jnp.float32)

        @pl.when(s == k_steps)
        def _():
            ys_ref[...] = part

        @pl.when(s > k_steps)
        def _():
            ys_ref[...] = ys_ref[...] + part

        @pl.when(s == 2 * k_steps - 1)
        def _epilogue():
            x = xs_ref[...]
            rs_val = rs_ref[meta_ref[t]]
            y = (ys_ref[...] + b2_ref[0]) * rs_val + x
            logits = lax.dot_general(x, wg_ref[...], (((1,), (1,)), ((), ())),
                                     preferred_element_type=jnp.float32,
                                     precision=lax.Precision.HIGHEST)
            logits = logits + bg_ref[...]
            e = logits.shape[-1]
            iota = lax.broadcasted_iota(jnp.int32, logits.shape, 1)
            m1 = jnp.max(logits, axis=-1, keepdims=True)
            i1 = jnp.min(jnp.where(logits == m1, iota, e), axis=-1,
                         keepdims=True)
            m2 = jnp.max(jnp.where(iota == i1, -jnp.inf, logits), axis=-1,
                         keepdims=True)
            ee = jnp.exp(m2 - m1)
            ssum = 1.0 + ee
            wsum = 1.0 / ssum + ee / ssum
            ys_ref[...] = y * wsum


def _grouped_ffn(xs, meta, W1, b1, ln_g, ln_b, W2, b2, rs, Wg, bg,
                 t_max, interpret=False):
    n_pad, d = xs.shape
    e, dff, _ = W1.shape
    k_steps = dff // KB
    grid = (t_max, 2 * k_steps)

    t_last = t_max

    def _row(t, m):
        return jnp.where(t < m[t_last], t, jnp.maximum(m[t_last] - 1, 0))

    def _k1(t, s, m):
        return jnp.where(t < m[t_last], jnp.minimum(s, k_steps - 1),
                         k_steps - 1)

    def _k2(t, s, m):
        return jnp.where(t < m[t_last], jnp.maximum(s - k_steps, 0),
                         k_steps - 1)

    grid_spec = pltpu.PrefetchScalarGridSpec(
        num_scalar_prefetch=1,
        grid=grid,
        in_specs=[
            pl.BlockSpec((TM, d), lambda t, s, m: (_row(t, m), 0)),
            pl.BlockSpec((1, KB, d), lambda t, s, m: (m[t], _k1(t, s, m), 0)),
            pl.BlockSpec((1, d, KB), lambda t, s, m: (m[t], 0, _k2(t, s, m))),
            pl.BlockSpec((1, 1, 1, KB),
                         lambda t, s, m: (m[t], _k1(t, s, m), 0, 0)),
            pl.BlockSpec((1, 1, 1, KB),
                         lambda t, s, m: (m[t], _k2(t, s, m), 0, 0)),
            pl.BlockSpec((1, 1, 1, KB),
                         lambda t, s, m: (m[t], _k2(t, s, m), 0, 0)),
            pl.BlockSpec((1, 1, d), lambda t, s, m: (m[t], 0, 0)),
            pl.BlockSpec((e, d), lambda t, s, m: (0, 0)),
            pl.BlockSpec((1, e), lambda t, s, m: (0, 0)),
            pl.BlockSpec(memory_space=pltpu.SMEM),
        ],
        out_specs=pl.BlockSpec((TM, d), lambda t, s, m: (_row(t, m), 0)),
        scratch_shapes=[
            pltpu.VMEM((k_steps, TM, KB), jnp.bfloat16),
            pltpu.VMEM((TM, d), jnp.bfloat16),
            pltpu.VMEM((TM, 128), jnp.float32),
        ],
    )
    body = functools.partial(_ffn_body, k_steps=k_steps, t_max=t_max,
                             dff=dff)
    return pl.pallas_call(
        body,
        grid_spec=grid_spec,
        out_shape=jax.ShapeDtypeStruct((n_pad, d), jnp.float32),
        interpret=interpret,
    )(meta, xs, W1, W2,
      b1.reshape(e, k_steps, 1, KB), ln_g.reshape(e, k_steps, 1, KB),
      ln_b.reshape(e, k_steps, 1, KB), b2.reshape(e, 1, d), Wg,
      bg.reshape(1, e), rs)


def kernel(x, W1, b1, ln_g, ln_b, W2, b2, rs, Wg, bg):
    bsz, seq, d = x.shape
    e = W1.shape[0]
    n = bsz * seq
    t_max = n // TM + e
    n_pad = t_max * TM
    xf = x.reshape(n, d)

    e_sel = _gate_experts(xf, Wg, bg)

    oh = (e_sel[:, None] == jnp.arange(e, dtype=jnp.int32)[None, :])
    csum = jnp.cumsum(oh.astype(jnp.int32), axis=0)
    counts = csum[-1]
    within = jnp.take_along_axis(csum, e_sel[:, None], axis=1)[:, 0] - 1
    tiles_e = (counts + TM - 1) // TM
    aligned = tiles_e * TM
    a_start = jnp.concatenate([jnp.zeros(1, jnp.int32),
                               jnp.cumsum(aligned)[:-1].astype(jnp.int32)])
    pos = a_start[e_sel] + within
    idxpad = (jnp.arange(n_pad, dtype=jnp.int32) % n).at[pos].set(
        jnp.arange(n, dtype=jnp.int32))
    tile_end = jnp.cumsum(tiles_e).astype(jnp.int32)
    total_tiles = tile_end[-1]
    tt = jnp.arange(t_max, dtype=jnp.int32)
    te = jnp.sum((tt[:, None] >= tile_end[None, :]).astype(jnp.int32), axis=1)
    te = jnp.minimum(te, e - 1)
    meta = jnp.concatenate([te, total_tiles[None]]).astype(jnp.int32)

    xs = _sc_gather_rows(xf, idxpad)
    ys = _grouped_ffn(xs, meta, W1, b1, ln_g, ln_b, W2, b2, rs, Wg, bg, t_max)
    out = _sc_gather_rows(ys, pos)
    return out.reshape(bsz, seq, d)


# speedup vs baseline: 1.0258x; 1.0258x over previous
import functools
import math

import jax
import jax.numpy as jnp
from jax import lax
from jax.experimental import pallas as pl
from jax.experimental.pallas import tpu as pltpu
from jax.experimental.pallas import tpu_sc as plsc


TM = 512
KB = 512
GATE_TB = 512


def _gate_experts(xf, Wg, bg):
    gate_logits = xf @ Wg.T + bg
    _, top_idx = lax.top_k(gate_logits, 2)
    return jnp.max(top_idx, axis=-1).astype(jnp.int32)


def _sc_gather_rows(table, idx):
    v, d = table.shape
    b = idx.shape[0]
    info = plsc.get_sparse_core_info()
    nw = info.num_cores * info.num_subcores
    assert b % (8 * nw) == 0 and d % info.num_lanes == 0
    b_per_w = b // nw
    ch = 32
    assert b_per_w % ch == 0
    mesh = plsc.VectorSubcoreMesh(core_axis_name="c", subcore_axis_name="s")

    @functools.partial(
        pl.kernel,
        mesh=mesh,
        out_type=jax.ShapeDtypeStruct((b, d), table.dtype),
        scratch_types=[
            pltpu.VMEM((ch,), jnp.int32),
            pltpu.VMEM((ch, d), table.dtype),
            pltpu.SemaphoreType.DMA,
        ],
    )
    def gather(table_hbm, idx_hbm, out_hbm, idx_v, rows_v, sem):
        wid = lax.axis_index("s") * info.num_cores + lax.axis_index("c")
        base = wid * b_per_w
        for c in range(b_per_w // ch):
            off = base + c * ch
            pltpu.sync_copy(idx_hbm.at[pl.ds(off, ch)], idx_v)
            pltpu.async_copy(table_hbm.at[idx_v], rows_v, sem).wait()
            pltpu.sync_copy(rows_v, out_hbm.at[pl.ds(off, ch)])

    return gather(table, idx)


def _ffn_body(meta_ref, xs_ref, w1_ref, w2_ref, b1_ref, g_ref, lb_ref,
              b2_ref, wg_ref, bg_ref, rs_ref, ys_ref, h_ref, xb_ref,
              st_ref, *, k_steps, t_max, dff):
    t = pl.program_id(0)
    s = pl.program_id(1)
    active = t < meta_ref[t_max]

    @pl.when(active & (s == 0))
    def _cast_x():
        xb_ref[...] = xs_ref[...].astype(jnp.bfloat16)

    @pl.when(active & (s < k_steps))
    def _phase1():
        w1 = w1_ref[0].astype(jnp.bfloat16)
        h = lax.dot_general(xb_ref[...], w1, (((1,), (1,)), ((), ())),
                            preferred_element_type=jnp.float32)
        h = h + b1_ref[0, 0]
        h_ref[s] = h.astype(jnp.bfloat16)
        rsum = jnp.sum(h, axis=-1, keepdims=True)
        rsq = jnp.sum(h * h, axis=-1, keepdims=True)

        @pl.when(s == 0)
        def _():
            st_ref[:, 0:1] = rsum
            st_ref[:, 1:2] = rsq

        @pl.when(s > 0)
        def _():
            st_ref[:, 0:1] = st_ref[:, 0:1] + rsum
            st_ref[:, 1:2] = st_ref[:, 1:2] + rsq

    @pl.when(active & (s >= k_steps))
    def _phase2():
        k = s - k_steps
        mu = st_ref[:, 0:1] / dff
        var = st_ref[:, 1:2] / dff - mu * mu
        hn = (h_ref[k].astype(jnp.float32) - mu) / jnp.sqrt(var + 1e-5)
        hn = hn * g_ref[0, 0] + lb_ref[0, 0]
        gh = hn * 0.5 * (1.0 + lax.erf(hn * (1.0 / math.sqrt(2.0))))
        w2 = w2_ref[0].astype(jnp.bfloat16)
        part = lax.dot_general(gh.astype(jnp.bfloat16), w2,
                               (((1,), (1,)), ((), ())),
                               preferred_element_type=jnp.float32)

        @pl.when(s == k_steps)
        def _():
            ys_ref[...] = part

        @pl.when(s > k_steps)
        def _():
            ys_ref[...] = ys_ref[...] + part

        @pl.when(s == 2 * k_steps - 1)
        def _epilogue():
            x = xs_ref[...]
            rs_val = rs_ref[meta_ref[t]]
            y = (ys_ref[...] + b2_ref[0]) * rs_val + x
            logits = lax.dot_general(x, wg_ref[...], (((1,), (1,)), ((), ())),
                                     preferred_element_type=jnp.float32,
                                     precision=lax.Precision.HIGHEST)
            logits = logits + bg_ref[...]
            e = logits.shape[-1]
            iota = lax.broadcasted_iota(jnp.int32, logits.shape, 1)
            m1 = jnp.max(logits, axis=-1, keepdims=True)
            i1 = jnp.min(jnp.where(logits == m1, iota, e), axis=-1,
                         keepdims=True)
            m2 = jnp.max(jnp.where(iota == i1, -jnp.inf, logits), axis=-1,
                         keepdims=True)
            ee = jnp.exp(m2 - m1)
            ssum = 1.0 + ee
            wsum = 1.0 / ssum + ee / ssum
            ys_ref[...] = y * wsum


def _grouped_ffn(xs, meta, W1, b1, ln_g, ln_b, W2, b2, rs, Wg, bg,
                 t_max, interpret=False):
    n_pad, d = xs.shape
    e, dff, _ = W1.shape
    k_steps = dff // KB
    grid = (t_max, 2 * k_steps)

    t_last = t_max

    def _row(t, m):
        return jnp.where(t < m[t_last], t, jnp.maximum(m[t_last] - 1, 0))

    def _k1(t, s, m):
        return jnp.where(t < m[t_last], jnp.minimum(s, k_steps - 1),
                         k_steps - 1)

    def _k2(t, s, m):
        return jnp.where(t < m[t_last], jnp.maximum(s - k_steps, 0),
                         k_steps - 1)

    grid_spec = pltpu.PrefetchScalarGridSpec(
        num_scalar_prefetch=1,
        grid=grid,
        in_specs=[
            pl.BlockSpec((TM, d), lambda t, s, m: (_row(t, m), 0)),
            pl.BlockSpec((1, KB, d), lambda t, s, m: (m[t], _k1(t, s, m), 0)),
            pl.BlockSpec((1, d, KB), lambda t, s, m: (m[t], 0, _k2(t, s, m))),
            pl.BlockSpec((1, 1, 1, KB),
                         lambda t, s, m: (m[t], _k1(t, s, m), 0, 0)),
            pl.BlockSpec((1, 1, 1, KB),
                         lambda t, s, m: (m[t], _k2(t, s, m), 0, 0)),
            pl.BlockSpec((1, 1, 1, KB),
                         lambda t, s, m: (m[t], _k2(t, s, m), 0, 0)),
            pl.BlockSpec((1, 1, d), lambda t, s, m: (m[t], 0, 0)),
            pl.BlockSpec((e, d), lambda t, s, m: (0, 0)),
            pl.BlockSpec((1, e), lambda t, s, m: (0, 0)),
            pl.BlockSpec(memory_space=pltpu.SMEM),
        ],
        out_specs=pl.BlockSpec((TM, d), lambda t, s, m: (_row(t, m), 0)),
        scratch_shapes=[
            pltpu.VMEM((k_steps, TM, KB), jnp.bfloat16),
            pltpu.VMEM((TM, d), jnp.bfloat16),
            pltpu.VMEM((TM, 128), jnp.float32),
        ],
    )
    body = functools.partial(_ffn_body, k_steps=k_steps, t_max=t_max,
                             dff=dff)
    return pl.pallas_call(
        body,
        grid_spec=grid_spec,
        out_shape=jax.ShapeDtypeStruct((n_pad, d), jnp.float32),
        interpret=interpret,
    )(meta, xs, W1, W2,
      b1.reshape(e, k_steps, 1, KB), ln_g.reshape(e, k_steps, 1, KB),
      ln_b.reshape(e, k_steps, 1, KB), b2.reshape(e, 1, d), Wg,
      bg.reshape(1, e), rs)


def kernel(x, W1, b1, ln_g, ln_b, W2, b2, rs, Wg, bg):
    bsz, seq, d = x.shape
    e = W1.shape[0]
    n = bsz * seq
    t_max = n // TM + e
    n_pad = -(-(t_max * TM) // 1024) * 1024
    xf = x.reshape(n, d)

    e_sel = _gate_experts(xf, Wg, bg)

    oh = (e_sel[:, None] == jnp.arange(e, dtype=jnp.int32)[None, :])
    csum = jnp.cumsum(oh.astype(jnp.int32), axis=0)
    counts = csum[-1]
    within = jnp.take_along_axis(csum, e_sel[:, None], axis=1)[:, 0] - 1
    tiles_e = (counts + TM - 1) // TM
    aligned = tiles_e * TM
    a_start = jnp.concatenate([jnp.zeros(1, jnp.int32),
                               jnp.cumsum(aligned)[:-1].astype(jnp.int32)])
    pos = a_start[e_sel] + within
    idxpad = (jnp.arange(n_pad, dtype=jnp.int32) % n).at[pos].set(
        jnp.arange(n, dtype=jnp.int32))
    tile_end = jnp.cumsum(tiles_e).astype(jnp.int32)
    total_tiles = tile_end[-1]
    tt = jnp.arange(t_max, dtype=jnp.int32)
    te = jnp.sum((tt[:, None] >= tile_end[None, :]).astype(jnp.int32), axis=1)
    te = jnp.minimum(te, e - 1)
    meta = jnp.concatenate([te, total_tiles[None]]).astype(jnp.int32)

    xs = _sc_gather_rows(xf, idxpad)
    ys = _grouped_ffn(xs, meta, W1, b1, ln_g, ln_b, W2, b2, rs, Wg, bg, t_max)
    out = _sc_gather_rows(ys, pos)
    return out.reshape(bsz, seq, d)
